# Initial kernel scaffold; baseline (speedup 1.0000x reference)
import jax
import jax.numpy as jnp
from jax.experimental import pallas as pl


def kernel(output, mask, target):
    raise NotImplementedError("write your pallas kernel here")


# speedup vs baseline: 1.2484x; 1.2484x over previous
import numpy as np
import jax
import jax.numpy as jnp
from jax.experimental import pallas as pl
from jax.experimental.pallas import tpu as pltpu

H, W = 32, 64
N = H * W
NPAD = 2176
RB = 128
NRB = NPAD // RB
BT = 8
FOV_UP_DEG, FOV_DOWN_DEG = 3.0, -25.0
MASK_THRESHOLD = 0.5
SENT = 1000.0


def _dirs_np():
    fov_up = FOV_UP_DEG * np.pi / 180.0
    fov_down = FOV_DOWN_DEG * np.pi / 180.0
    fov = abs(fov_up) + abs(fov_down)
    proj_y = (np.arange(H, dtype=np.float32) + 0.5) / H
    proj_x = (np.arange(W, dtype=np.float32) + 0.5) / W
    pitch = (1.0 - proj_y) * fov - abs(fov_down)
    yaw = (2.0 * proj_x - 1.0) * np.pi
    pitch = pitch[:, None]
    yaw = yaw[None, :]
    dx = np.cos(pitch) * np.cos(yaw)
    dy = np.cos(pitch) * np.sin(yaw)
    dz = np.sin(pitch) * np.ones_like(yaw)
    dirs = np.stack([np.broadcast_to(dx, (H, W)),
                     np.broadcast_to(dy, (H, W)),
                     np.broadcast_to(dz, (H, W))], axis=-1).astype(np.float32)
    return dirs.reshape(N, 3)


_DIRS = _dirs_np()


def _build_body(out_ref, mask_ref, tr_ref, tx_ref, ty_ref, tz_ref,
                dx_ref, dy_ref, dz_ref,
                ax_ref, ay_ref, az_ref, bx_ref, by_ref, bz_ref):
    r = jnp.where(mask_ref[...] > MASK_THRESHOLD, out_ref[...], -1.0)
    valid = r > 0.0
    ax = jnp.where(valid, r * dx_ref[...], SENT)
    ay = jnp.where(valid, r * dy_ref[...], SENT)
    az = jnp.where(valid, r * dz_ref[...], SENT)
    tvalid = tr_ref[...] >= 0.0
    bx = jnp.where(tvalid, tx_ref[...], SENT)
    by = jnp.where(tvalid, ty_ref[...], SENT)
    bz = jnp.where(tvalid, tz_ref[...], SENT)
    for dst, src in ((ax_ref, ax), (ay_ref, ay), (az_ref, az),
                     (bx_ref, bx), (by_ref, by), (bz_ref, bz)):
        dst[:, :N] = src
        dst[:, N:] = jnp.full((BT, NPAD - N), SENT, jnp.float32)


def _pair_body(axc, ayc, azc, bxr, byr, bzr, out_ref, colmin, acc):
    rb = pl.program_id(1)
    a_x = axc[0]
    a_y = ayc[0]
    a_z = azc[0]
    b_x = bxr[0]
    b_y = byr[0]
    b_z = bzr[0]
    dx = a_x - b_x
    dy = a_y - b_y
    dz = a_z - b_z
    d = dx * dx + dy * dy + dz * dz
    rmin = jnp.min(d, axis=1)
    s1 = jnp.sum(rmin)
    c1 = jnp.sum((rmin > 0.0).astype(jnp.float32))
    cm = jnp.min(d, axis=0, keepdims=True)

    @pl.when(rb == 0)
    def _():
        colmin[...] = cm
        acc[0] = s1
        acc[1] = c1

    @pl.when(rb > 0)
    def _():
        colmin[...] = jnp.minimum(colmin[...], cm)
        acc[0] = acc[0] + s1
        acc[1] = acc[1] + c1

    @pl.when(rb == NRB - 1)
    def _():
        cmf = colmin[...]
        s2 = jnp.sum(cmf)
        c2 = jnp.sum((cmf > 0.0).astype(jnp.float32))
        out_ref[...] = jnp.full((1, 1, 1), acc[0] / acc[1] + s2 / c2,
                                jnp.float32)


def _build_points(out2, mask2, tr, tx, ty, tz):
    dx = _DIRS[:, 0].reshape(1, N)
    dy = _DIRS[:, 1].reshape(1, N)
    dz = _DIRS[:, 2].reshape(1, N)
    plane = jax.ShapeDtypeStruct((BT, NPAD), jnp.float32)
    return pl.pallas_call(
        _build_body,
        out_shape=(plane,) * 6,
    )(out2, mask2, tr, tx, ty, tz,
      jnp.asarray(dx), jnp.asarray(dy), jnp.asarray(dz))


def _pairwise(ax, ay, az, bx, by, bz):
    a_spec = pl.BlockSpec((1, RB, 1), lambda f, rb: (f, rb, 0))
    b_spec = pl.BlockSpec((1, 1, NPAD), lambda f, rb: (f, 0, 0))
    return pl.pallas_call(
        _pair_body,
        grid=(BT, NRB),
        in_specs=[a_spec, a_spec, a_spec, b_spec, b_spec, b_spec],
        out_specs=pl.BlockSpec((1, 1, 1), lambda f, rb: (f, 0, 0)),
        out_shape=jax.ShapeDtypeStruct((BT, 1, 1), jnp.float32),
        scratch_shapes=[
            pltpu.VMEM((1, NPAD), jnp.float32),
            pltpu.SMEM((2,), jnp.float32),
        ],
    )(ax.reshape(BT, NPAD, 1), ay.reshape(BT, NPAD, 1),
      az.reshape(BT, NPAD, 1),
      bx.reshape(BT, 1, NPAD), by.reshape(BT, 1, NPAD),
      bz.reshape(BT, 1, NPAD))


def kernel(output, mask, target):
    B, T = output.shape[0], output.shape[1]
    out2 = output.reshape(BT, N)
    mask2 = mask.reshape(BT, N)
    tr = target[:, :, 0].reshape(BT, N)
    tx = target[:, :, 1].reshape(BT, N)
    ty = target[:, :, 2].reshape(BT, N)
    tz = target[:, :, 3].reshape(BT, N)
    ax, ay, az, bx, by, bz = _build_points(out2, mask2, tr, tx, ty, tz)
    dc = _pairwise(ax, ay, az, bx, by, bz).reshape(BT)
    ct = dc.reshape(T, B)
    return (jnp.mean(ct, axis=1), ct)


# speedup vs baseline: 1.6345x; 1.3092x over previous
import numpy as np
import jax
import jax.numpy as jnp
from jax.experimental import pallas as pl
from jax.experimental.pallas import tpu as pltpu

H, W = 32, 64
N = H * W
NPAD = 2176
RB = 272
NRB = NPAD // RB
BT = 8
FOV_UP_DEG, FOV_DOWN_DEG = 3.0, -25.0
MASK_THRESHOLD = 0.5
SENT = 1000.0


def _dirs_np():
    fov_up = FOV_UP_DEG * np.pi / 180.0
    fov_down = FOV_DOWN_DEG * np.pi / 180.0
    fov = abs(fov_up) + abs(fov_down)
    proj_y = (np.arange(H, dtype=np.float32) + 0.5) / H
    proj_x = (np.arange(W, dtype=np.float32) + 0.5) / W
    pitch = (1.0 - proj_y) * fov - abs(fov_down)
    yaw = (2.0 * proj_x - 1.0) * np.pi
    pitch = pitch[:, None]
    yaw = yaw[None, :]
    dx = np.cos(pitch) * np.cos(yaw)
    dy = np.cos(pitch) * np.sin(yaw)
    dz = np.sin(pitch) * np.ones_like(yaw)
    dirs = np.stack([np.broadcast_to(dx, (H, W)),
                     np.broadcast_to(dy, (H, W)),
                     np.broadcast_to(dz, (H, W))], axis=-1).astype(np.float32)
    return dirs.reshape(N, 3)


_DIRS = _dirs_np()


def _build_body(out_ref, mask_ref, tr_ref, tx_ref, ty_ref, tz_ref,
                dx_ref, dy_ref, dz_ref,
                ax_ref, ay_ref, az_ref, bx_ref, by_ref, bz_ref):
    r = jnp.where(mask_ref[...] > MASK_THRESHOLD, out_ref[...], -1.0)
    valid = r > 0.0
    ax = jnp.where(valid, r * dx_ref[...], SENT)
    ay = jnp.where(valid, r * dy_ref[...], SENT)
    az = jnp.where(valid, r * dz_ref[...], SENT)
    tvalid = tr_ref[...] >= 0.0
    bx = jnp.where(tvalid, tx_ref[...], SENT)
    by = jnp.where(tvalid, ty_ref[...], SENT)
    bz = jnp.where(tvalid, tz_ref[...], SENT)
    for dst, src in ((ax_ref, ax), (ay_ref, ay), (az_ref, az),
                     (bx_ref, bx), (by_ref, by), (bz_ref, bz)):
        dst[:, :N] = src
        dst[:, N:] = jnp.full((BT, NPAD - N), SENT, jnp.float32)


def _pair_body(axc, ayc, azc, bxr, byr, bzr, out_ref, colmin, acc):
    rb = pl.program_id(1)
    a_x = axc[0]
    a_y = ayc[0]
    a_z = azc[0]
    b_x = bxr[0]
    b_y = byr[0]
    b_z = bzr[0]
    dx = a_x - b_x
    dy = a_y - b_y
    dz = a_z - b_z
    d = dx * dx + dy * dy + dz * dz
    rmin = jnp.min(d, axis=1)
    s1 = jnp.sum(rmin)
    c1 = jnp.sum((rmin > 0.0).astype(jnp.float32))
    cm = jnp.min(d, axis=0, keepdims=True)

    @pl.when(rb == 0)
    def _():
        colmin[...] = cm
        acc[0] = s1
        acc[1] = c1

    @pl.when(rb > 0)
    def _():
        colmin[...] = jnp.minimum(colmin[...], cm)
        acc[0] = acc[0] + s1
        acc[1] = acc[1] + c1

    @pl.when(rb == NRB - 1)
    def _():
        cmf = colmin[...]
        s2 = jnp.sum(cmf)
        c2 = jnp.sum((cmf > 0.0).astype(jnp.float32))
        out_ref[...] = jnp.full((1, 1, 1), acc[0] / acc[1] + s2 / c2,
                                jnp.float32)


def _build_points(out2, mask2, tr, tx, ty, tz):
    dx = _DIRS[:, 0].reshape(1, N)
    dy = _DIRS[:, 1].reshape(1, N)
    dz = _DIRS[:, 2].reshape(1, N)
    plane = jax.ShapeDtypeStruct((BT, NPAD), jnp.float32)
    return pl.pallas_call(
        _build_body,
        out_shape=(plane,) * 6,
    )(out2, mask2, tr, tx, ty, tz,
      jnp.asarray(dx), jnp.asarray(dy), jnp.asarray(dz))


def _pairwise(ax, ay, az, bx, by, bz):
    a_spec = pl.BlockSpec((1, RB, 1), lambda f, rb: (f, rb, 0))
    b_spec = pl.BlockSpec((1, 1, NPAD), lambda f, rb: (f, 0, 0))
    return pl.pallas_call(
        _pair_body,
        grid=(BT, NRB),
        in_specs=[a_spec, a_spec, a_spec, b_spec, b_spec, b_spec],
        out_specs=pl.BlockSpec((1, 1, 1), lambda f, rb: (f, 0, 0)),
        out_shape=jax.ShapeDtypeStruct((BT, 1, 1), jnp.float32),
        scratch_shapes=[
            pltpu.VMEM((1, NPAD), jnp.float32),
            pltpu.SMEM((2,), jnp.float32),
        ],
    )(ax.reshape(BT, NPAD, 1), ay.reshape(BT, NPAD, 1),
      az.reshape(BT, NPAD, 1),
      bx.reshape(BT, 1, NPAD), by.reshape(BT, 1, NPAD),
      bz.reshape(BT, 1, NPAD))


def kernel(output, mask, target):
    B, T = output.shape[0], output.shape[1]
    out2 = output.reshape(BT, N)
    mask2 = mask.reshape(BT, N)
    tr = target[:, :, 0].reshape(BT, N)
    tx = target[:, :, 1].reshape(BT, N)
    ty = target[:, :, 2].reshape(BT, N)
    tz = target[:, :, 3].reshape(BT, N)
    ax, ay, az, bx, by, bz = _build_points(out2, mask2, tr, tx, ty, tz)
    dc = _pairwise(ax, ay, az, bx, by, bz).reshape(BT)
    ct = dc.reshape(T, B)
    return (jnp.mean(ct, axis=1), ct)


# speedup vs baseline: 1.8132x; 1.1093x over previous
import numpy as np
import jax
import jax.numpy as jnp
from jax.experimental import pallas as pl
from jax.experimental.pallas import tpu as pltpu

H, W = 32, 64
N = H * W
NPAD = 2176
RB = 544
NRB = NPAD // RB
BT = 8
FOV_UP_DEG, FOV_DOWN_DEG = 3.0, -25.0
MASK_THRESHOLD = 0.5
SENT = 1000.0


def _dirs_np():
    fov_up = FOV_UP_DEG * np.pi / 180.0
    fov_down = FOV_DOWN_DEG * np.pi / 180.0
    fov = abs(fov_up) + abs(fov_down)
    proj_y = (np.arange(H, dtype=np.float32) + 0.5) / H
    proj_x = (np.arange(W, dtype=np.float32) + 0.5) / W
    pitch = (1.0 - proj_y) * fov - abs(fov_down)
    yaw = (2.0 * proj_x - 1.0) * np.pi
    pitch = pitch[:, None]
    yaw = yaw[None, :]
    dx = np.cos(pitch) * np.cos(yaw)
    dy = np.cos(pitch) * np.sin(yaw)
    dz = np.sin(pitch) * np.ones_like(yaw)
    dirs = np.stack([np.broadcast_to(dx, (H, W)),
                     np.broadcast_to(dy, (H, W)),
                     np.broadcast_to(dz, (H, W))], axis=-1).astype(np.float32)
    return dirs.reshape(N, 3)


_DIRS = _dirs_np()


def _build_body(out_ref, mask_ref, tr_ref, tx_ref, ty_ref, tz_ref,
                dx_ref, dy_ref, dz_ref,
                ax_ref, ay_ref, az_ref, bx_ref, by_ref, bz_ref):
    r = jnp.where(mask_ref[...] > MASK_THRESHOLD, out_ref[...], -1.0)
    valid = r > 0.0
    ax = jnp.where(valid, r * dx_ref[...], SENT)
    ay = jnp.where(valid, r * dy_ref[...], SENT)
    az = jnp.where(valid, r * dz_ref[...], SENT)
    tvalid = tr_ref[...] >= 0.0
    bx = jnp.where(tvalid, tx_ref[...], SENT)
    by = jnp.where(tvalid, ty_ref[...], SENT)
    bz = jnp.where(tvalid, tz_ref[...], SENT)
    for dst, src in ((ax_ref, ax), (ay_ref, ay), (az_ref, az),
                     (bx_ref, bx), (by_ref, by), (bz_ref, bz)):
        dst[:, :N] = src
        dst[:, N:] = jnp.full((BT, NPAD - N), SENT, jnp.float32)


def _pair_body(axc, ayc, azc, bxr, byr, bzr, out_ref, colmin, acc):
    rb = pl.program_id(1)
    a_x = axc[0]
    a_y = ayc[0]
    a_z = azc[0]
    b_x = bxr[0]
    b_y = byr[0]
    b_z = bzr[0]
    dx = a_x - b_x
    dy = a_y - b_y
    dz = a_z - b_z
    d = dx * dx + dy * dy + dz * dz
    rmin = jnp.min(d, axis=1)
    s1 = jnp.sum(rmin)
    c1 = jnp.sum((rmin > 0.0).astype(jnp.float32))
    cm = jnp.min(d, axis=0, keepdims=True)

    @pl.when(rb == 0)
    def _():
        colmin[...] = cm
        acc[0] = s1
        acc[1] = c1

    @pl.when(rb > 0)
    def _():
        colmin[...] = jnp.minimum(colmin[...], cm)
        acc[0] = acc[0] + s1
        acc[1] = acc[1] + c1

    @pl.when(rb == NRB - 1)
    def _():
        cmf = colmin[...]
        s2 = jnp.sum(cmf)
        c2 = jnp.sum((cmf > 0.0).astype(jnp.float32))
        out_ref[...] = jnp.full((1, 1, 1), acc[0] / acc[1] + s2 / c2,
                                jnp.float32)


def _build_points(out2, mask2, tr, tx, ty, tz):
    dx = _DIRS[:, 0].reshape(1, N)
    dy = _DIRS[:, 1].reshape(1, N)
    dz = _DIRS[:, 2].reshape(1, N)
    plane = jax.ShapeDtypeStruct((BT, NPAD), jnp.float32)
    return pl.pallas_call(
        _build_body,
        out_shape=(plane,) * 6,
    )(out2, mask2, tr, tx, ty, tz,
      jnp.asarray(dx), jnp.asarray(dy), jnp.asarray(dz))


def _pairwise(ax, ay, az, bx, by, bz):
    a_spec = pl.BlockSpec((1, RB, 1), lambda f, rb: (f, rb, 0))
    b_spec = pl.BlockSpec((1, 1, NPAD), lambda f, rb: (f, 0, 0))
    return pl.pallas_call(
        _pair_body,
        grid=(BT, NRB),
        in_specs=[a_spec, a_spec, a_spec, b_spec, b_spec, b_spec],
        out_specs=pl.BlockSpec((1, 1, 1), lambda f, rb: (f, 0, 0)),
        out_shape=jax.ShapeDtypeStruct((BT, 1, 1), jnp.float32),
        scratch_shapes=[
            pltpu.VMEM((1, NPAD), jnp.float32),
            pltpu.SMEM((2,), jnp.float32),
        ],
    )(ax.reshape(BT, NPAD, 1), ay.reshape(BT, NPAD, 1),
      az.reshape(BT, NPAD, 1),
      bx.reshape(BT, 1, NPAD), by.reshape(BT, 1, NPAD),
      bz.reshape(BT, 1, NPAD))


def kernel(output, mask, target):
    B, T = output.shape[0], output.shape[1]
    out2 = output.reshape(BT, N)
    mask2 = mask.reshape(BT, N)
    tr = target[:, :, 0].reshape(BT, N)
    tx = target[:, :, 1].reshape(BT, N)
    ty = target[:, :, 2].reshape(BT, N)
    tz = target[:, :, 3].reshape(BT, N)
    ax, ay, az, bx, by, bz = _build_points(out2, mask2, tr, tx, ty, tz)
    dc = _pairwise(ax, ay, az, bx, by, bz).reshape(BT)
    ct = dc.reshape(T, B)
    return (jnp.mean(ct, axis=1), ct)


# speedup vs baseline: 1.8805x; 1.0371x over previous
import numpy as np
import jax
import jax.numpy as jnp
from jax.experimental import pallas as pl
from jax.experimental.pallas import tpu as pltpu

H, W = 32, 64
N = H * W
NPAD = 2176
RB = 1088
NRB = NPAD // RB
BT = 8
FOV_UP_DEG, FOV_DOWN_DEG = 3.0, -25.0
MASK_THRESHOLD = 0.5
SENT = 1000.0


def _dirs_np():
    fov_up = FOV_UP_DEG * np.pi / 180.0
    fov_down = FOV_DOWN_DEG * np.pi / 180.0
    fov = abs(fov_up) + abs(fov_down)
    proj_y = (np.arange(H, dtype=np.float32) + 0.5) / H
    proj_x = (np.arange(W, dtype=np.float32) + 0.5) / W
    pitch = (1.0 - proj_y) * fov - abs(fov_down)
    yaw = (2.0 * proj_x - 1.0) * np.pi
    pitch = pitch[:, None]
    yaw = yaw[None, :]
    dx = np.cos(pitch) * np.cos(yaw)
    dy = np.cos(pitch) * np.sin(yaw)
    dz = np.sin(pitch) * np.ones_like(yaw)
    dirs = np.stack([np.broadcast_to(dx, (H, W)),
                     np.broadcast_to(dy, (H, W)),
                     np.broadcast_to(dz, (H, W))], axis=-1).astype(np.float32)
    return dirs.reshape(N, 3)


_DIRS = _dirs_np()


def _build_body(out_ref, mask_ref, tr_ref, tx_ref, ty_ref, tz_ref,
                dx_ref, dy_ref, dz_ref,
                ax_ref, ay_ref, az_ref, bx_ref, by_ref, bz_ref):
    r = jnp.where(mask_ref[...] > MASK_THRESHOLD, out_ref[...], -1.0)
    valid = r > 0.0
    ax = jnp.where(valid, r * dx_ref[...], SENT)
    ay = jnp.where(valid, r * dy_ref[...], SENT)
    az = jnp.where(valid, r * dz_ref[...], SENT)
    tvalid = tr_ref[...] >= 0.0
    bx = jnp.where(tvalid, tx_ref[...], SENT)
    by = jnp.where(tvalid, ty_ref[...], SENT)
    bz = jnp.where(tvalid, tz_ref[...], SENT)
    for dst, src in ((ax_ref, ax), (ay_ref, ay), (az_ref, az),
                     (bx_ref, bx), (by_ref, by), (bz_ref, bz)):
        dst[:, :N] = src
        dst[:, N:] = jnp.full((BT, NPAD - N), SENT, jnp.float32)


def _pair_body(axc, ayc, azc, bxr, byr, bzr, out_ref, colmin, acc):
    rb = pl.program_id(1)
    a_x = axc[0]
    a_y = ayc[0]
    a_z = azc[0]
    b_x = bxr[0]
    b_y = byr[0]
    b_z = bzr[0]
    dx = a_x - b_x
    dy = a_y - b_y
    dz = a_z - b_z
    d = dx * dx + dy * dy + dz * dz
    rmin = jnp.min(d, axis=1)
    s1 = jnp.sum(rmin)
    c1 = jnp.sum((rmin > 0.0).astype(jnp.float32))
    cm = jnp.min(d, axis=0, keepdims=True)

    @pl.when(rb == 0)
    def _():
        colmin[...] = cm
        acc[0] = s1
        acc[1] = c1

    @pl.when(rb > 0)
    def _():
        colmin[...] = jnp.minimum(colmin[...], cm)
        acc[0] = acc[0] + s1
        acc[1] = acc[1] + c1

    @pl.when(rb == NRB - 1)
    def _():
        cmf = colmin[...]
        s2 = jnp.sum(cmf)
        c2 = jnp.sum((cmf > 0.0).astype(jnp.float32))
        out_ref[...] = jnp.full((1, 1, 1), acc[0] / acc[1] + s2 / c2,
                                jnp.float32)


def _build_points(out2, mask2, tr, tx, ty, tz):
    dx = _DIRS[:, 0].reshape(1, N)
    dy = _DIRS[:, 1].reshape(1, N)
    dz = _DIRS[:, 2].reshape(1, N)
    plane = jax.ShapeDtypeStruct((BT, NPAD), jnp.float32)
    return pl.pallas_call(
        _build_body,
        out_shape=(plane,) * 6,
    )(out2, mask2, tr, tx, ty, tz,
      jnp.asarray(dx), jnp.asarray(dy), jnp.asarray(dz))


def _pairwise(ax, ay, az, bx, by, bz):
    a_spec = pl.BlockSpec((1, RB, 1), lambda f, rb: (f, rb, 0))
    b_spec = pl.BlockSpec((1, 1, NPAD), lambda f, rb: (f, 0, 0))
    return pl.pallas_call(
        _pair_body,
        grid=(BT, NRB),
        in_specs=[a_spec, a_spec, a_spec, b_spec, b_spec, b_spec],
        out_specs=pl.BlockSpec((1, 1, 1), lambda f, rb: (f, 0, 0)),
        out_shape=jax.ShapeDtypeStruct((BT, 1, 1), jnp.float32),
        scratch_shapes=[
            pltpu.VMEM((1, NPAD), jnp.float32),
            pltpu.SMEM((2,), jnp.float32),
        ],
    )(ax.reshape(BT, NPAD, 1), ay.reshape(BT, NPAD, 1),
      az.reshape(BT, NPAD, 1),
      bx.reshape(BT, 1, NPAD), by.reshape(BT, 1, NPAD),
      bz.reshape(BT, 1, NPAD))


def kernel(output, mask, target):
    B, T = output.shape[0], output.shape[1]
    out2 = output.reshape(BT, N)
    mask2 = mask.reshape(BT, N)
    tr = target[:, :, 0].reshape(BT, N)
    tx = target[:, :, 1].reshape(BT, N)
    ty = target[:, :, 2].reshape(BT, N)
    tz = target[:, :, 3].reshape(BT, N)
    ax, ay, az, bx, by, bz = _build_points(out2, mask2, tr, tx, ty, tz)
    dc = _pairwise(ax, ay, az, bx, by, bz).reshape(BT)
    ct = dc.reshape(T, B)
    return (jnp.mean(ct, axis=1), ct)
